# baseline (speedup 1.0000x reference)
import functools

import jax
import jax.numpy as jnp
from jax import lax
from jax.experimental import pallas as pl
from jax.experimental.pallas import tpu as pltpu
from jax.experimental.pallas import tpu_sc as plsc


def _sc_gather_sum(table, flat_idx, n_groups, rows_per_group, d):
    info = plsc.get_sparse_core_info()
    nc = info.num_cores
    mesh = plsc.VectorSubcoreMesh(core_axis_name="c", subcore_axis_name="s")

    @functools.partial(
        pl.kernel,
        mesh=mesh,
        out_type=jax.ShapeDtypeStruct((n_groups, d), jnp.float32),
        scratch_types=[
            pltpu.VMEM((rows_per_group,), jnp.int32),
            pltpu.VMEM((rows_per_group, d), jnp.float32),
            pltpu.VMEM((d,), jnp.float32),
            pltpu.SemaphoreType.DMA,
        ],
    )
    def gather_kernel(table_hbm, idx_hbm, out_hbm, idx_v, rows_v, acc_v, sem):
        wid = lax.axis_index("s") * nc + lax.axis_index("c")

        @pl.when(wid < n_groups)
        def _():
            base = wid * rows_per_group
            pltpu.sync_copy(idx_hbm.at[pl.ds(base, rows_per_group)], idx_v)
            pltpu.async_copy(table_hbm.at[idx_v], rows_v, sem).wait()

            def body(i, carry):
                sl = pl.ds(i * 16, 16)
                acc = rows_v[0, sl]
                for r in range(1, rows_per_group):
                    acc = acc + rows_v[r, sl]
                acc_v[sl] = acc
                return carry

            lax.fori_loop(0, d // 16, body, 0)
            pltpu.sync_copy(acc_v, out_hbm.at[wid])

    return gather_kernel(table, flat_idx)


def _attn_body(e_ref, h_ref, o_ref):
    hb = h_ref[0]
    e12 = e_ref[0]
    s = lax.dot_general(
        hb, e12, (((1,), (1,)), ((), ())),
        preferred_element_type=jnp.float32,
    )
    p = jnp.exp(s - jnp.max(s, axis=0, keepdims=True))
    w = p / jnp.sum(p, axis=0, keepdims=True)
    o_ref[0, 0] = 0.5 * jnp.sum(w, axis=1)


def kernel(input_embed_M, e1_index, e2_index):
    B, S, D = input_embed_M.shape
    K = e1_index.shape[-1]
    eidx = jnp.concatenate(
        [e1_index.astype(jnp.int32), e2_index.astype(jnp.int32)], axis=1
    )
    flat_idx = (
        eidx + (jnp.arange(B, dtype=jnp.int32) * S)[:, None]
    ).reshape(B * 2 * K)
    table = input_embed_M.reshape(B * S, D)
    embeds = _sc_gather_sum(table, flat_idx, B * 2, K, D)
    e = embeds.reshape(B, 2, D)
    out = pl.pallas_call(
        _attn_body,
        grid=(B,),
        in_specs=[
            pl.BlockSpec((1, 2, D), lambda b: (b, 0, 0)),
            pl.BlockSpec((1, S, D), lambda b: (b, 0, 0)),
        ],
        out_specs=pl.BlockSpec((1, 1, S), lambda b: (b, 0, 0)),
        out_shape=jax.ShapeDtypeStruct((B, 1, S), jnp.float32),
    )(e, input_embed_M)
    return out[:, 0, :]


# speedup vs baseline: 1.4309x; 1.4309x over previous
import jax
import jax.numpy as jnp
from jax import lax
from jax.experimental import pallas as pl
from jax.experimental.pallas import tpu as pltpu

_DC = 4


def _attn_body(idx_ref, h_ref, o_ref, acc_ref):
    K = idx_ref.shape[-1] // 2
    dc = pl.program_id(1)
    hb = h_ref[0]
    e1 = h_ref[0, idx_ref[0, 0, 0], :]
    e2 = h_ref[0, idx_ref[0, 0, K], :]
    for k in range(1, K):
        e1 = e1 + h_ref[0, idx_ref[0, 0, k], :]
        e2 = e2 + h_ref[0, idx_ref[0, 0, K + k], :]
    e12 = jnp.stack([e1, e2], axis=0)
    s = lax.dot_general(
        hb, e12, (((1,), (1,)), ((), ())),
        preferred_element_type=jnp.float32,
    )

    @pl.when(dc == 0)
    def _():
        acc_ref[...] = s

    @pl.when(dc != 0)
    def _():
        acc_ref[...] = acc_ref[...] + s

    @pl.when(dc == _DC - 1)
    def _():
        t = acc_ref[...]
        p = jnp.exp(t - jnp.max(t, axis=0, keepdims=True))
        w = p / jnp.sum(p, axis=0, keepdims=True)
        o_ref[0, 0] = 0.5 * jnp.sum(w, axis=1)


def kernel(input_embed_M, e1_index, e2_index):
    B, S, D = input_embed_M.shape
    K = e1_index.shape[-1]
    eidx = jnp.concatenate(
        [e1_index.astype(jnp.int32), e2_index.astype(jnp.int32)], axis=1
    ).reshape(B, 1, 2 * K)
    out = pl.pallas_call(
        _attn_body,
        grid=(B, _DC),
        in_specs=[
            pl.BlockSpec((1, 1, 2 * K), lambda b, dc: (b, 0, 0),
                         memory_space=pltpu.SMEM),
            pl.BlockSpec((1, S, D // _DC), lambda b, dc: (b, 0, dc)),
        ],
        out_specs=pl.BlockSpec((1, 1, S), lambda b, dc: (b, 0, 0)),
        out_shape=jax.ShapeDtypeStruct((B, 1, S), jnp.float32),
        scratch_shapes=[pltpu.VMEM((S, 2), jnp.float32)],
    )(eidx, input_embed_M)
    return out[:, 0, :]


# speedup vs baseline: 1.5940x; 1.1140x over previous
import jax
import jax.numpy as jnp
from jax import lax
from jax.experimental import pallas as pl
from jax.experimental.pallas import tpu as pltpu

_CS = 256
_NBUF = 3


def _attn_body(idx_ref, h_ref, o_ref, bufs, rows, sacc, csem, rsem):
    B, S, D = h_ref.shape
    K2 = idx_ref.shape[-1]
    K = K2 // 2
    C = S // _CS
    nchunks = B * C

    def chunk_copy(i):
        b, c = divmod(i, C)
        return pltpu.make_async_copy(
            h_ref.at[b, pl.ds(c * _CS, _CS), :], bufs.at[i % _NBUF],
            csem.at[i % _NBUF])

    def row_copies(b):
        hs = []
        for g in range(K2):
            hs.append(pltpu.make_async_copy(
                h_ref.at[b, idx_ref[b, g]], rows.at[b * K2 + g], rsem))
        return hs

    row_handles = {0: row_copies(0)}
    for h in row_handles[0]:
        h.start()
    handles = []
    for i in range(min(_NBUF, nchunks)):
        handles.append(chunk_copy(i))
        handles[i].start()

    e12 = None
    for i in range(nchunks):
        b, c = divmod(i, C)
        if c == 0:
            for h in row_handles[b]:
                h.wait()
            e1 = rows[b * K2, :]
            e2 = rows[b * K2 + K, :]
            for k in range(1, K):
                e1 = e1 + rows[b * K2 + k, :]
                e2 = e2 + rows[b * K2 + K + k, :]
            e12 = jnp.stack([e1, e2], axis=0)
            if b + 1 < B:
                row_handles[b + 1] = row_copies(b + 1)
                for h in row_handles[b + 1]:
                    h.start()
        handles[i].wait()
        s = lax.dot_general(
            bufs[i % _NBUF], e12, (((1,), (1,)), ((), ())),
            preferred_element_type=jnp.float32,
        )
        sacc[pl.ds(c * _CS, _CS), :] = s
        if i + _NBUF < nchunks:
            handles.append(chunk_copy(i + _NBUF))
            handles[i + _NBUF].start()
        if c == C - 1:
            t = sacc[...]
            p = jnp.exp(t - jnp.max(t, axis=0, keepdims=True))
            w = p / jnp.sum(p, axis=0, keepdims=True)
            o_ref[b, :] = 0.5 * jnp.sum(w, axis=1)


def kernel(input_embed_M, e1_index, e2_index):
    B, S, D = input_embed_M.shape
    K = e1_index.shape[-1]
    eidx = jnp.concatenate(
        [e1_index.astype(jnp.int32), e2_index.astype(jnp.int32)], axis=1
    )
    return pl.pallas_call(
        _attn_body,
        in_specs=[
            pl.BlockSpec(memory_space=pltpu.SMEM),
            pl.BlockSpec(memory_space=pltpu.MemorySpace.HBM),
        ],
        out_specs=pl.BlockSpec(memory_space=pltpu.VMEM),
        out_shape=jax.ShapeDtypeStruct((B, S), jnp.float32),
        scratch_shapes=[
            pltpu.VMEM((_NBUF, _CS, D), jnp.float32),
            pltpu.VMEM((B * 2 * K, D), jnp.float32),
            pltpu.VMEM((S, 2), jnp.float32),
            pltpu.SemaphoreType.DMA((_NBUF,)),
            pltpu.SemaphoreType.DMA,
        ],
    )(eidx, input_embed_M)
